# baseline (speedup 1.0000x reference)
import functools

import numpy as np
import jax
import jax.numpy as jnp
from jax import lax
from jax.experimental import pallas as pl
from jax.experimental.pallas import tpu as pltpu
from jax.experimental.pallas import tpu_sc as plsc


def _dense_body(xn, xg, wn, bn, wg, bg, on, og):
    on[...] = jnp.dot(xn[...], wn[...], preferred_element_type=jnp.float32) + bn[...]
    og[...] = jnp.dot(xg[...], wg[...], preferred_element_type=jnp.float32) + bg[...]


def _dense_tokens(X_num, X_geo, W_num, b_num, W_geo, b_geo):
    B, NUM = X_num.shape
    NGEO = X_geo.shape[1]
    D = W_num.shape[1]
    bs = 2048
    out = pl.pallas_call(
        _dense_body,
        grid=(B // bs,),
        in_specs=[
            pl.BlockSpec((bs, NUM), lambda i: (i, 0)),
            pl.BlockSpec((bs, NGEO), lambda i: (i, 0)),
            pl.BlockSpec((NUM, D), lambda i: (0, 0)),
            pl.BlockSpec((1, D), lambda i: (0, 0)),
            pl.BlockSpec((NGEO, D), lambda i: (0, 0)),
            pl.BlockSpec((1, D), lambda i: (0, 0)),
        ],
        out_specs=[
            pl.BlockSpec((bs, D), lambda i: (i, 0)),
            pl.BlockSpec((bs, D), lambda i: (i, 0)),
        ],
        out_shape=[
            jax.ShapeDtypeStruct((B, D), jnp.float32),
            jax.ShapeDtypeStruct((B, D), jnp.float32),
        ],
    )(X_num, X_geo, W_num, b_num.reshape(1, D), W_geo, b_geo.reshape(1, D))
    return out


@functools.lru_cache(maxsize=None)
def _make_sc_tokenizer(B, NCAT, VOCAB, D, NC, NS, L):
    NT = NCAT + 2
    NW = NC * NS
    RPW = B // NW
    C = 64
    NCH = RPW // C
    CN = C * NCAT
    G = CN // 128
    assert CN % 128 == 0 and RPW % C == 0 and B % NW == 0 and 2 * C == 128

    mesh = plsc.VectorSubcoreMesh(core_axis_name="c", subcore_axis_name="s")

    @functools.partial(
        pl.kernel,
        out_type=jax.ShapeDtypeStruct((B * NT, D), jnp.float32),
        mesh=mesh,
        compiler_params=pltpu.CompilerParams(use_tc_tiling_on_sc=False),
        scratch_types=[
            pltpu.VMEM((CN,), jnp.int32),
            pltpu.VMEM((G, 128), jnp.int32),
            pltpu.VMEM((CN,), jnp.int32),
            pltpu.VMEM((G, 128), jnp.int32),
            pltpu.VMEM((1, 128), jnp.int32),
            pltpu.VMEM((1, 128), jnp.int32),
            pltpu.VMEM((G, 128, D), jnp.float32),
            pltpu.VMEM((2 * C, D), jnp.float32),
            pltpu.SemaphoreType.DMA,
        ],
    )
    def sc_tok(tabf, xcat1, numt, geot, opat_h, dpat_h, ddpat_h, outf,
               idx1, dst2, opv, dpv, ddpv, ddst, rows, dstage, sem):
        wid = lax.axis_index("s") * NC + lax.axis_index("c")
        pltpu.sync_copy(opat_h, opv)
        pltpu.sync_copy(dpat_h, dpv)
        pltpu.sync_copy(ddpat_h, ddpv)

        def chunk_body(c, carry):
            b0 = wid * RPW + c * C
            pltpu.sync_copy(xcat1.at[pl.ds(b0 * NCAT, CN)], idx1)
            basev = jnp.full((L,), b0 * NT, dtype=jnp.int32)
            for g in range(G):
                for h in range(128 // L):
                    fsl = pl.ds((g * 128 + h * L), L)
                    sl = (g, pl.ds(h * L, L))
                    idx1[fsl] = idx1[fsl] + opv[fsl]
                    dst2[sl] = dpv[sl] + basev
            gathers = [
                pltpu.async_copy(tabf.at[idx1.at[pl.ds(g * 128, 128)]], rows.at[g], sem)
                for g in range(G)
            ]
            pltpu.sync_copy(numt.at[pl.ds(b0, C), :], dstage.at[pl.ds(0, C), :])
            pltpu.sync_copy(geot.at[pl.ds(b0, C), :], dstage.at[pl.ds(C, C), :])
            for h in range(128 // L):
                sl = (0, pl.ds(h * L, L))
                ddst[sl] = ddpv[sl] + basev
            for cp in gathers:
                cp.wait()
            scatters = [
                pltpu.async_copy(rows.at[g], outf.at[dst2.at[g]], sem)
                for g in range(G)
            ]
            scatters.append(pltpu.async_copy(dstage, outf.at[ddst.at[0]], sem))
            for cp in scatters:
                cp.wait()
            return carry

        lax.fori_loop(0, NCH, chunk_body, 0)

    i = np.arange(CN)
    opat = ((i % NCAT) * VOCAB).astype(np.int32)
    dpat = ((i // NCAT) * NT + 1 + (i % NCAT)).astype(np.int32).reshape(G, 128)
    j = np.arange(2 * C)
    ddpat = np.where(j < C, j * NT, (j - C) * NT + NT - 1).astype(np.int32)
    ddpat = ddpat.reshape(1, 2 * C)
    return sc_tok, jnp.asarray(opat), jnp.asarray(dpat), jnp.asarray(ddpat), NT


def kernel(X_num, X_cat, X_geo, W_num, b_num, tables, W_geo, b_geo):
    B = X_num.shape[0]
    NCAT, VOCAB, D = tables.shape
    try:
        info = plsc.get_sparse_core_info()
        NC, NS, L = info.num_cores, info.num_subcores, info.num_lanes
    except Exception:
        NC, NS, L = 2, 16, 16

    numt, geot = _dense_tokens(X_num, X_geo, W_num, b_num, W_geo, b_geo)
    sc_tok, opat, dpat, ddpat, NT = _make_sc_tokenizer(B, NCAT, VOCAB, D, NC, NS, L)

    tabf = tables.reshape(NCAT * VOCAB, D)
    xcat1 = X_cat.reshape(B * NCAT)
    outf = sc_tok(tabf, xcat1, numt, geot, opat, dpat, ddpat)
    return outf.reshape(B, NT, D)
